# baseline (speedup 1.0000x reference)
import functools

import jax
import jax.numpy as jnp
from jax import lax
from jax.experimental import pallas as pl
from jax.experimental.pallas import tpu as pltpu
from jax.experimental.pallas import tpu_sc as plsc

NC = 2
NS = 16
NW = NC * NS
CHUNK = 128
NBUF = 8


@functools.partial(jax.jit, static_argnames=("n_pad", "c", "k_chunks"))
def _edge_scatter(y, src3, dst3, zeros, *, n_pad, c, k_chunks):
    rows_per_tile = n_pad // NS
    mesh = plsc.VectorSubcoreMesh(core_axis_name="c", subcore_axis_name="s")
    kg = k_chunks // NBUF

    @functools.partial(
        pl.kernel,
        out_type=jax.ShapeDtypeStruct((NC, n_pad, c), jnp.float32),
        mesh=mesh,
        scratch_types=[
            pltpu.VMEM((kg, NBUF * CHUNK), jnp.int32),
            pltpu.VMEM((kg, NBUF * CHUNK), jnp.int32),
            pltpu.VMEM((2, NBUF * CHUNK, c), jnp.float32),
            pltpu.VMEM_SHARED((n_pad, c), jnp.float32),
            pltpu.SemaphoreType.DMA((2,)),
            pltpu.SemaphoreType.DMA((2,)),
        ],
        compiler_params=pltpu.CompilerParams(use_tc_tiling_on_sc=False),
    )
    def k(y_hbm, src_hbm, dst_hbm, z_hbm, out_hbm, src_v, dst_v, rows_v,
          agg_sh, sem, sem_s):
        cid = lax.axis_index("c")
        sid = lax.axis_index("s")
        wid = cid * NS + sid
        pltpu.sync_copy(src_hbm.at[wid], src_v)
        pltpu.sync_copy(dst_hbm.at[wid], dst_v)
        r0 = sid * rows_per_tile
        pltpu.sync_copy(z_hbm.at[pl.ds(r0, rows_per_tile)],
                        agg_sh.at[pl.ds(r0, rows_per_tile)])
        plsc.subcore_barrier()

        def g_wait(g, par):
            pltpu.make_async_copy(y_hbm.at[src_v.at[g]], rows_v.at[par],
                                  sem.at[par]).wait()

        def s_wait(par):
            pltpu.make_async_copy(y_hbm.at[src_v.at[0]], rows_v.at[par],
                                  sem_s.at[par]).wait()

        pltpu.async_copy(y_hbm.at[src_v.at[0]], rows_v.at[0], sem.at[0])

        def group(g, carry):
            par = lax.rem(g, 2)
            npar = lax.rem(g + 1, 2)
            g_wait(g, par)

            @pl.when(g + 1 < kg)
            def _():
                @pl.when(g >= 1)
                def _():
                    s_wait(npar)
                pltpu.async_copy(y_hbm.at[src_v.at[g + 1]], rows_v.at[npar],
                                 sem.at[npar])

            pltpu.async_copy(rows_v.at[par], agg_sh.at[dst_v.at[g]],
                             sem_s.at[par], add=True)
            return carry

        lax.fori_loop(0, kg, group, 0)
        s_wait((kg - 1) % 2)
        plsc.subcore_barrier()
        pltpu.sync_copy(agg_sh.at[pl.ds(r0, rows_per_tile)],
                        out_hbm.at[cid, pl.ds(r0, rows_per_tile)])

    return k(y, src3, dst3, zeros)


def _mm_first(x8, bd_nb, bd_self, r_out):
    def body(x_ref, a_ref, b_ref, y_ref, s_ref):
        xx = x_ref[...]
        y_ref[...] = jnp.dot(xx, a_ref[...],
                             preferred_element_type=jnp.float32,
                             precision=lax.Precision.HIGHEST)
        s_ref[...] = jnp.dot(xx, b_ref[...],
                             preferred_element_type=jnp.float32,
                             precision=lax.Precision.HIGHEST)

    return pl.pallas_call(
        body,
        out_shape=(jax.ShapeDtypeStruct((r_out, 128), jnp.float32),
                   jax.ShapeDtypeStruct((r_out, 128), jnp.float32)),
    )(x8, bd_nb, bd_self)


def _fused(parts_f, s_f, gl, bl, fold, res_f, bd_nb, bd_self, n, r_real):
    r = s_f.shape[0]
    inv_n = 1.0 / n
    fuse_mm = bd_nb is not None

    def body(*refs):
        it = iter(refs)
        p_ref, s_ref, g_ref, b_ref, f_ref = (next(it) for _ in range(5))
        r_ref = next(it) if res_f is not None else None
        wa_ref = next(it) if fuse_mm else None
        wb_ref = next(it) if fuse_mm else None
        h_ref = next(it)
        pre = p_ref[0, :r] + p_ref[1, :r] + s_ref[...]
        f = f_ref[...]
        m128 = jnp.sum(pre, axis=0, keepdims=True) * inv_n
        mc = jnp.dot(m128, f, preferred_element_type=jnp.float32,
                             precision=lax.Precision.HIGHEST)
        mx = lax.dot_general(mc, f, (((1,), (1,)), ((), ())),
                             preferred_element_type=jnp.float32,
                             precision=lax.Precision.HIGHEST)
        rows = lax.broadcasted_iota(jnp.int32, (r, 1), 0)
        live = rows < r_real
        d = jnp.where(live, pre - mx, 0.0)
        v128 = jnp.sum(d * d, axis=0, keepdims=True) * inv_n
        vc = jnp.dot(v128, f, preferred_element_type=jnp.float32,
                             precision=lax.Precision.HIGHEST)
        vx = lax.dot_general(vc, f, (((1,), (1,)), ((), ())),
                             preferred_element_type=jnp.float32,
                             precision=lax.Precision.HIGHEST)
        hn = d * lax.rsqrt(vx + 1e-3) * g_ref[...] + b_ref[...]
        if r_ref is not None:
            hn = hn + r_ref[...]
        h = jnp.maximum(hn, 0.0)
        h = jnp.where(live, h, 0.0)
        h_ref[...] = h
        if fuse_mm:
            y_ref, s2_ref = next(it), next(it)
            y_ref[...] = jnp.dot(h, wa_ref[...],
                                 preferred_element_type=jnp.float32,
                             precision=lax.Precision.HIGHEST)
            s2_ref[...] = jnp.dot(h, wb_ref[...],
                                  preferred_element_type=jnp.float32,
                             precision=lax.Precision.HIGHEST)

    args = [parts_f, s_f, gl, bl, fold]
    if res_f is not None:
        args.append(res_f)
    outs = [jax.ShapeDtypeStruct((r, 128), jnp.float32)]
    if fuse_mm:
        args += [bd_nb, bd_self]
        x_cols = bd_nb.shape[1]
        outs += [jax.ShapeDtypeStruct((r, x_cols), jnp.float32),
                 jax.ShapeDtypeStruct((r, x_cols), jnp.float32)]
    return pl.pallas_call(body, out_shape=tuple(outs))(*args)


def kernel(x, edge_index, Win_nb, Win_self, g_in, b_in, S1_nb, S1_self,
           S1_g, S1_b, Wd_nb, Wd_self, g_d, b_d, S2_nb, S2_self, S2_g, S2_b):
    n = x.shape[0]
    e = edge_index.shape[1]
    n_flat = -(-n // 64) * 64
    n_sc = -(-(n_flat + 1) // 128) * 128
    k_chunks = -(-e // (NW * CHUNK * NBUF)) * NBUF
    e_pad = NW * k_chunks * CHUNK
    kg = k_chunks // NBUF

    src = edge_index[0].astype(jnp.int32)
    dst = edge_index[1].astype(jnp.int32)
    src3 = jnp.concatenate(
        [src, jnp.zeros((e_pad - e,), jnp.int32)]).reshape(
            NW, kg, NBUF * CHUNK)
    dst3 = jnp.concatenate(
        [dst, jnp.full((e_pad - e,), n_flat, jnp.int32)]).reshape(
            NW, kg, NBUF * CHUNK)

    z16 = jnp.zeros((n_sc, 16), jnp.float32)
    z32 = jnp.zeros((n_sc, 32), jnp.float32)

    def fold_mat(c):
        return (jnp.arange(128)[:, None] % c ==
                jnp.arange(c)[None, :]).astype(jnp.float32)

    f16, f32m = fold_mat(16), fold_mat(32)

    def bd(w):
        return jnp.kron(jnp.eye(128 // w.shape[0], dtype=jnp.float32), w)

    def lane(v):
        return jnp.tile(v, 128 // v.shape[0])[None, :]

    def sc_pass(y_f, c):
        y_std = y_f.reshape(n_flat, c)
        z = z16 if c == 16 else z32
        parts = _edge_scatter(y_std, src3, dst3, z,
                              n_pad=n_sc, c=c, k_chunks=k_chunks)
        return parts.reshape(NC, n_sc * c // 128, 128)

    def layer(y_f, s_f, c, g, b, res, wnb, wself):
        parts_f = sc_pass(y_f, c)
        bd_nb = bd(wnb) if wnb is not None else None
        bd_self = bd(wself) if wself is not None else None
        fold = f16 if c == 16 else f32m
        outs = _fused(parts_f, s_f, lane(g), lane(b), fold, res,
                      bd_nb, bd_self, n, n * c // 128)
        if wnb is None:
            return outs[0], None, None
        h_f, y2, s2 = outs
        c2 = wnb.shape[1]
        r2 = n_flat * c2 // 128
        return h_f, y2.reshape(r2, 128), s2.reshape(r2, 128)

    r16 = n_flat * 16 // 128
    x8 = jnp.pad(x.reshape(n // 8, 8 * x.shape[1]),
                 ((0, r16 - n // 8), (0, 0)))
    bd1n = jnp.kron(jnp.eye(8, dtype=jnp.float32), Win_nb)
    bd1s = jnp.kron(jnp.eye(8, dtype=jnp.float32), Win_self)
    y, s = _mm_first(x8, bd1n, bd1s, r16)

    h1, y, s = layer(y, s, 16, g_in, b_in, None, S1_nb[0, 0], S1_self[0, 0])
    o1, y, s = layer(y, s, 16, S1_g[0, 0], S1_b[0, 0], None,
                     S1_nb[0, 1], S1_self[0, 1])
    h2, y, s = layer(y, s, 16, S1_g[0, 1], S1_b[0, 1], h1,
                     S1_nb[1, 0], S1_self[1, 0])
    o2, y, s = layer(y, s, 16, S1_g[1, 0], S1_b[1, 0], None,
                     S1_nb[1, 1], S1_self[1, 1])
    h3, y, s = layer(y, s, 16, S1_g[1, 1], S1_b[1, 1], h2, Wd_nb, Wd_self)
    h4, y, s = layer(y, s, 32, g_d, b_d, None, S2_nb[0, 0], S2_self[0, 0])
    o3, y, s = layer(y, s, 32, S2_g[0, 0], S2_b[0, 0], None,
                     S2_nb[0, 1], S2_self[0, 1])
    h5, y, s = layer(y, s, 32, S2_g[0, 1], S2_b[0, 1], h4,
                     S2_nb[1, 0], S2_self[1, 0])
    o4, y, s = layer(y, s, 32, S2_g[1, 0], S2_b[1, 0], None,
                     S2_nb[1, 1], S2_self[1, 1])
    h6, _, _ = layer(y, s, 32, S2_g[1, 1], S2_b[1, 1], h5, None, None)
    return h6.reshape(n_flat, 32)[:n]


# speedup vs baseline: 1.0208x; 1.0208x over previous
import functools

import jax
import jax.numpy as jnp
from jax import lax
from jax.experimental import pallas as pl
from jax.experimental.pallas import tpu as pltpu
from jax.experimental.pallas import tpu_sc as plsc

NC = 2
NS = 16
NW = NC * NS
CHUNK = 128
NBUF = 8


@functools.partial(jax.jit, static_argnames=("n_pad", "c", "k_chunks"))
def _edge_scatter(y, src3, dst3, zeros, *, n_pad, c, k_chunks):
    rows_per_tile = n_pad // NS
    mesh = plsc.VectorSubcoreMesh(core_axis_name="c", subcore_axis_name="s")
    kg = k_chunks // NBUF

    @functools.partial(
        pl.kernel,
        out_type=jax.ShapeDtypeStruct((NC, n_pad, c), jnp.float32),
        mesh=mesh,
        scratch_types=[
            pltpu.VMEM((kg, NBUF * CHUNK), jnp.int32),
            pltpu.VMEM((kg, NBUF * CHUNK), jnp.int32),
            pltpu.VMEM((2, NBUF * CHUNK, c), jnp.float32),
            pltpu.VMEM_SHARED((n_pad, c), jnp.float32),
            pltpu.SemaphoreType.DMA((2,)),
            pltpu.SemaphoreType.DMA((2,)),
        ],
        compiler_params=pltpu.CompilerParams(use_tc_tiling_on_sc=False),
    )
    def k(y_hbm, src_hbm, dst_hbm, z_hbm, out_hbm, src_v, dst_v, rows_v,
          agg_sh, sem, sem_s):
        cid = lax.axis_index("c")
        sid = lax.axis_index("s")
        wid = cid * NS + sid
        pltpu.sync_copy(src_hbm.at[wid], src_v)
        pltpu.sync_copy(dst_hbm.at[wid], dst_v)
        r0 = sid * rows_per_tile
        pltpu.sync_copy(z_hbm.at[pl.ds(r0, rows_per_tile)],
                        agg_sh.at[pl.ds(r0, rows_per_tile)])
        plsc.subcore_barrier()

        def g_wait(g, par):
            pltpu.make_async_copy(y_hbm.at[src_v.at[g]], rows_v.at[par],
                                  sem.at[par]).wait()

        def s_wait(par):
            pltpu.make_async_copy(y_hbm.at[src_v.at[0]], rows_v.at[par],
                                  sem_s.at[par]).wait()

        pltpu.async_copy(y_hbm.at[src_v.at[0]], rows_v.at[0], sem.at[0])

        def group(g, carry):
            par = lax.rem(g, 2)
            npar = lax.rem(g + 1, 2)
            g_wait(g, par)

            @pl.when(g + 1 < kg)
            def _():
                @pl.when(g >= 1)
                def _():
                    s_wait(npar)
                pltpu.async_copy(y_hbm.at[src_v.at[g + 1]], rows_v.at[npar],
                                 sem.at[npar])

            pltpu.async_copy(rows_v.at[par], agg_sh.at[dst_v.at[g]],
                             sem_s.at[par], add=True)
            return carry

        lax.fori_loop(0, kg, group, 0)
        s_wait((kg - 1) % 2)
        plsc.subcore_barrier()
        pltpu.sync_copy(agg_sh.at[pl.ds(r0, rows_per_tile)],
                        out_hbm.at[cid, pl.ds(r0, rows_per_tile)])

    return k(y, src3, dst3, zeros)


def _mm_first(x8, bd_nb, bd_self, r_out):
    def body(x_ref, a_ref, b_ref, y_ref, s_ref):
        xx = x_ref[...]
        y_ref[...] = jnp.dot(xx, a_ref[...],
                             preferred_element_type=jnp.float32)
        s_ref[...] = jnp.dot(xx, b_ref[...],
                             preferred_element_type=jnp.float32)

    return pl.pallas_call(
        body,
        out_shape=(jax.ShapeDtypeStruct((r_out, 128), jnp.float32),
                   jax.ShapeDtypeStruct((r_out, 128), jnp.float32)),
    )(x8, bd_nb, bd_self)


def _fused(parts_f, s_f, gl, bl, fold, res_f, bd_nb, bd_self, n, r_real):
    r = s_f.shape[0]
    inv_n = 1.0 / n
    fuse_mm = bd_nb is not None

    def body(*refs):
        it = iter(refs)
        p_ref, s_ref, g_ref, b_ref, f_ref = (next(it) for _ in range(5))
        r_ref = next(it) if res_f is not None else None
        wa_ref = next(it) if fuse_mm else None
        wb_ref = next(it) if fuse_mm else None
        h_ref = next(it)
        pre = p_ref[0, :r] + p_ref[1, :r] + s_ref[...]
        f = f_ref[...]
        m128 = jnp.sum(pre, axis=0, keepdims=True) * inv_n
        mc = jnp.dot(m128, f, preferred_element_type=jnp.float32,
                             precision=lax.Precision.HIGHEST)
        mx = lax.dot_general(mc, f, (((1,), (1,)), ((), ())),
                             preferred_element_type=jnp.float32,
                             precision=lax.Precision.HIGHEST)
        rows = lax.broadcasted_iota(jnp.int32, (r, 1), 0)
        live = rows < r_real
        d = jnp.where(live, pre - mx, 0.0)
        v128 = jnp.sum(d * d, axis=0, keepdims=True) * inv_n
        vc = jnp.dot(v128, f, preferred_element_type=jnp.float32,
                             precision=lax.Precision.HIGHEST)
        vx = lax.dot_general(vc, f, (((1,), (1,)), ((), ())),
                             preferred_element_type=jnp.float32,
                             precision=lax.Precision.HIGHEST)
        hn = d * lax.rsqrt(vx + 1e-3) * g_ref[...] + b_ref[...]
        if r_ref is not None:
            hn = hn + r_ref[...]
        h = jnp.maximum(hn, 0.0)
        h = jnp.where(live, h, 0.0)
        h_ref[...] = h
        if fuse_mm:
            y_ref, s2_ref = next(it), next(it)
            y_ref[...] = jnp.dot(h, wa_ref[...],
                                 preferred_element_type=jnp.float32)
            s2_ref[...] = jnp.dot(h, wb_ref[...],
                                  preferred_element_type=jnp.float32)

    args = [parts_f, s_f, gl, bl, fold]
    if res_f is not None:
        args.append(res_f)
    outs = [jax.ShapeDtypeStruct((r, 128), jnp.float32)]
    if fuse_mm:
        args += [bd_nb, bd_self]
        x_cols = bd_nb.shape[1]
        outs += [jax.ShapeDtypeStruct((r, x_cols), jnp.float32),
                 jax.ShapeDtypeStruct((r, x_cols), jnp.float32)]
    return pl.pallas_call(body, out_shape=tuple(outs))(*args)


def kernel(x, edge_index, Win_nb, Win_self, g_in, b_in, S1_nb, S1_self,
           S1_g, S1_b, Wd_nb, Wd_self, g_d, b_d, S2_nb, S2_self, S2_g, S2_b):
    n = x.shape[0]
    e = edge_index.shape[1]
    n_flat = -(-n // 64) * 64
    n_sc = -(-(n_flat + 1) // 128) * 128
    k_chunks = -(-e // (NW * CHUNK * NBUF)) * NBUF
    e_pad = NW * k_chunks * CHUNK
    kg = k_chunks // NBUF

    src = edge_index[0].astype(jnp.int32)
    dst = edge_index[1].astype(jnp.int32)
    src3 = jnp.concatenate(
        [src, jnp.zeros((e_pad - e,), jnp.int32)]).reshape(
            NW, kg, NBUF * CHUNK)
    dst3 = jnp.concatenate(
        [dst, jnp.full((e_pad - e,), n_flat, jnp.int32)]).reshape(
            NW, kg, NBUF * CHUNK)

    z16 = jnp.zeros((n_sc, 16), jnp.float32)
    z32 = jnp.zeros((n_sc, 32), jnp.float32)

    def fold_mat(c):
        return (jnp.arange(128)[:, None] % c ==
                jnp.arange(c)[None, :]).astype(jnp.float32)

    f16, f32m = fold_mat(16), fold_mat(32)

    def bd(w):
        return jnp.kron(jnp.eye(128 // w.shape[0], dtype=jnp.float32), w)

    def lane(v):
        return jnp.tile(v, 128 // v.shape[0])[None, :]

    def sc_pass(y_f, c):
        y_std = y_f.reshape(n_flat, c)
        z = z16 if c == 16 else z32
        parts = _edge_scatter(y_std, src3, dst3, z,
                              n_pad=n_sc, c=c, k_chunks=k_chunks)
        return parts.reshape(NC, n_sc * c // 128, 128)

    def layer(y_f, s_f, c, g, b, res, wnb, wself):
        parts_f = sc_pass(y_f, c)
        bd_nb = bd(wnb) if wnb is not None else None
        bd_self = bd(wself) if wself is not None else None
        fold = f16 if c == 16 else f32m
        outs = _fused(parts_f, s_f, lane(g), lane(b), fold, res,
                      bd_nb, bd_self, n, n * c // 128)
        if wnb is None:
            return outs[0], None, None
        h_f, y2, s2 = outs
        c2 = wnb.shape[1]
        r2 = n_flat * c2 // 128
        return h_f, y2.reshape(r2, 128), s2.reshape(r2, 128)

    r16 = n_flat * 16 // 128
    x8 = jnp.pad(x.reshape(n // 8, 8 * x.shape[1]),
                 ((0, r16 - n // 8), (0, 0)))
    bd1n = jnp.kron(jnp.eye(8, dtype=jnp.float32), Win_nb)
    bd1s = jnp.kron(jnp.eye(8, dtype=jnp.float32), Win_self)
    y, s = _mm_first(x8, bd1n, bd1s, r16)

    h1, y, s = layer(y, s, 16, g_in, b_in, None, S1_nb[0, 0], S1_self[0, 0])
    o1, y, s = layer(y, s, 16, S1_g[0, 0], S1_b[0, 0], None,
                     S1_nb[0, 1], S1_self[0, 1])
    h2, y, s = layer(y, s, 16, S1_g[0, 1], S1_b[0, 1], h1,
                     S1_nb[1, 0], S1_self[1, 0])
    o2, y, s = layer(y, s, 16, S1_g[1, 0], S1_b[1, 0], None,
                     S1_nb[1, 1], S1_self[1, 1])
    h3, y, s = layer(y, s, 16, S1_g[1, 1], S1_b[1, 1], h2, Wd_nb, Wd_self)
    h4, y, s = layer(y, s, 32, g_d, b_d, None, S2_nb[0, 0], S2_self[0, 0])
    o3, y, s = layer(y, s, 32, S2_g[0, 0], S2_b[0, 0], None,
                     S2_nb[0, 1], S2_self[0, 1])
    h5, y, s = layer(y, s, 32, S2_g[0, 1], S2_b[0, 1], h4,
                     S2_nb[1, 0], S2_self[1, 0])
    o4, y, s = layer(y, s, 32, S2_g[1, 0], S2_b[1, 0], None,
                     S2_nb[1, 1], S2_self[1, 1])
    h6, _, _ = layer(y, s, 32, S2_g[1, 1], S2_b[1, 1], h5, None, None)
    return h6.reshape(n_flat, 32)[:n]
